# baseline (speedup 1.0000x reference)
import functools

import jax
import jax.numpy as jnp
from jax import lax
from jax.experimental import pallas as pl
from jax.experimental.pallas import tpu as pltpu
from jax.experimental.pallas import tpu_sc as plsc

HIDDEN = 32
OUT = 16


def _sc_gather_t(flat, idx, v):
    info = plsc.get_sparse_core_info()
    nc, ns = info.num_cores, info.num_subcores
    nw = nc * ns
    b = idx.shape[0]
    assert b % (8 * nw) == 0
    b_per_w = b // nw
    n_vec = b_per_w // 16
    mesh = plsc.VectorSubcoreMesh(core_axis_name="c", subcore_axis_name="s")

    @functools.partial(
        pl.kernel,
        mesh=mesh,
        out_type=jax.ShapeDtypeStruct((HIDDEN, b), jnp.float32),
        scratch_types=[
            pltpu.VMEM((b_per_w,), jnp.int32),
            pltpu.VMEM((HIDDEN, b_per_w), jnp.int32),
            pltpu.VMEM((HIDDEN, b_per_w), jnp.float32),
            pltpu.SemaphoreType.DMA,
        ],
        compiler_params=pltpu.CompilerParams(use_tc_tiling_on_sc=False),
    )
    def gather_kernel(flat_hbm, idx_hbm, out_hbm, idx_v, idxc_v, rows_v, sem):
        wid = lax.axis_index("s") * nc + lax.axis_index("c")
        base = wid * b_per_w
        pltpu.sync_copy(idx_hbm.at[pl.ds(base, b_per_w)], idx_v)

        def build(t, _):
            chunk = idx_v[pl.ds(t * 16, 16)]
            for c in range(HIDDEN):
                idxc_v[c, pl.ds(t * 16, 16)] = chunk + c * v
            return _

        lax.fori_loop(0, n_vec, build, None)

        copies = [
            pltpu.async_copy(flat_hbm.at[idxc_v.at[c]], rows_v.at[c], sem)
            for c in range(HIDDEN)
        ]
        for cp in copies:
            cp.wait()
        pltpu.sync_copy(rows_v, out_hbm.at[:, pl.ds(base, b_per_w)])

    return gather_kernel(flat, idx)


def _mlp_body(ht_ref, w_ref, b_ref, o_ref):
    h = jnp.maximum(ht_ref[...], 0.0)
    o_ref[...] = (
        lax.dot_general(
            w_ref[...], h, (((1,), (0,)), ((), ())),
            preferred_element_type=jnp.float32,
        )
        + b_ref[...]
    )


def kernel(x, emb, W2, b2):
    b = x.shape[0]
    v = emb.shape[0]
    idx = x.reshape(b).astype(jnp.int32)
    flat = emb.T.reshape(v * HIDDEN)
    ht = _sc_gather_t(flat, idx, v)
    yt = pl.pallas_call(
        _mlp_body,
        out_shape=jax.ShapeDtypeStruct((OUT, b), jnp.float32),
    )(ht, W2, b2.reshape(OUT, 1))
    return yt.T


# speedup vs baseline: 16.8446x; 16.8446x over previous
import functools

import jax
import jax.numpy as jnp
from jax import lax
from jax.experimental import pallas as pl
from jax.experimental.pallas import tpu as pltpu
from jax.experimental.pallas import tpu_sc as plsc

HIDDEN = 32
OUT = 16
CB = 1536
BLK = HIDDEN * CB
V = 1000000
NB = 999936 // CB
VMAIN = NB * CB
NTAIL = V - VMAIN
FLAT_N = NB * BLK


def _sc_detile(embt):
    info = plsc.get_sparse_core_info()
    nc, ns = info.num_cores, info.num_subcores
    nw = nc * ns
    mesh = plsc.VectorSubcoreMesh(core_axis_name="c", subcore_axis_name="s")

    @functools.partial(
        pl.kernel,
        mesh=mesh,
        out_type=jax.ShapeDtypeStruct((FLAT_N,), jnp.float32),
        scratch_types=[
            pltpu.VMEM((HIDDEN, CB), jnp.float32),
            pltpu.VMEM((HIDDEN, CB), jnp.float32),
            pltpu.SemaphoreType.DMA,
            pltpu.SemaphoreType.DMA,
            pltpu.SemaphoreType.DMA,
        ],
    )
    def detile_kernel(embt_hbm, flat_hbm, buf0, buf1, rsem0, rsem1, wsem):
        wid = lax.axis_index("s") * nc + lax.axis_index("c")
        n_my = (NB - wid + nw - 1) // nw

        def read_block(g, buf, rsem):
            r0 = pl.multiple_of(g * CB, 128)
            return pltpu.async_copy(embt_hbm.at[:, pl.ds(r0, CB)], buf, rsem)

        def write_block(g, buf, wsem):
            o0 = pl.multiple_of(g * BLK, 8)
            return [
                pltpu.async_copy(
                    buf.at[c], flat_hbm.at[pl.ds(o0 + c * CB, CB)], wsem
                )
                for c in range(HIDDEN)
            ]

        def body(i, _):
            g = wid + (2 * i) * nw
            r0 = read_block(g, buf0, rsem0)
            r1 = read_block(g + nw, buf1, rsem1)
            r0.wait()
            w0 = write_block(g, buf0, wsem)
            r1.wait()
            w1 = write_block(g + nw, buf1, wsem)
            for cp in w0:
                cp.wait()
            for cp in w1:
                cp.wait()
            return _

        lax.fori_loop(0, n_my // 2, body, None)

        @pl.when((n_my % 2) == 1)
        def _odd_tail():
            g = wid + (n_my - 1) * nw
            read_block(g, buf0, rsem0).wait()
            for cp in write_block(g, buf0, wsem):
                cp.wait()

    return detile_kernel(embt)


def _sc_gather_t(flat, pos0):
    info = plsc.get_sparse_core_info()
    nc, ns = info.num_cores, info.num_subcores
    nw = nc * ns
    b = pos0.shape[0]
    assert b % (8 * nw) == 0
    b_per_w = b // nw
    n_vec = b_per_w // 16
    mesh = plsc.VectorSubcoreMesh(core_axis_name="c", subcore_axis_name="s")

    @functools.partial(
        pl.kernel,
        mesh=mesh,
        out_type=jax.ShapeDtypeStruct((HIDDEN, b), jnp.float32),
        scratch_types=[
            pltpu.VMEM((b_per_w,), jnp.int32),
            pltpu.VMEM((HIDDEN, b_per_w), jnp.int32),
            pltpu.VMEM((HIDDEN, b_per_w), jnp.float32),
            pltpu.SemaphoreType.DMA,
        ],
        compiler_params=pltpu.CompilerParams(use_tc_tiling_on_sc=False),
    )
    def gather_kernel(flat_hbm, pos_hbm, out_hbm, pos_v, idxc_v, rows_v, sem):
        wid = lax.axis_index("s") * nc + lax.axis_index("c")
        base = wid * b_per_w
        pltpu.sync_copy(pos_hbm.at[pl.ds(base, b_per_w)], pos_v)

        def build(t, _):
            sl = pl.ds(t * 16, 16)
            acc = pos_v[sl]
            for c in range(HIDDEN):
                idxc_v[c, sl] = acc
                acc = acc + CB
            return _

        lax.fori_loop(0, n_vec, build, None)

        copies = [
            pltpu.async_copy(flat_hbm.at[idxc_v.at[c]], rows_v.at[c], sem)
            for c in range(HIDDEN)
        ]
        for cp in copies:
            cp.wait()
        pltpu.sync_copy(rows_v, out_hbm.at[:, pl.ds(base, b_per_w)])

    return gather_kernel(flat, pos0)


def _mlp_body(ht_ref, idx_ref, tail_ref, w_ref, b_ref, o_ref):
    ht = ht_ref[...]
    idx = idx_ref[...]
    tail_sel = idx - VMAIN
    onehot = jnp.where(
        lax.broadcasted_iota(jnp.int32, (NTAIL, idx.shape[1]), 0) == tail_sel,
        1.0,
        0.0,
    )
    htail = lax.dot_general(
        tail_ref[...], onehot, (((1,), (0,)), ((), ())),
        preferred_element_type=jnp.float32,
    )
    ht = jnp.where(idx >= VMAIN, htail, ht)
    h = jnp.maximum(ht, 0.0)
    o_ref[...] = (
        lax.dot_general(
            w_ref[...], h, (((1,), (0,)), ((), ())),
            preferred_element_type=jnp.float32,
        )
        + b_ref[...]
    )


def kernel(x, emb, W2, b2):
    b = x.shape[0]
    idx = x.reshape(b).astype(jnp.int32)
    flat = _sc_detile(emb.T)

    idx_c = jnp.minimum(idx, VMAIN - 1)
    g = idx_c // CB
    pos0 = g * BLK + (idx_c - g * CB)
    ht = _sc_gather_t(flat, pos0)

    tail_t = emb[VMAIN:].T
    yt = pl.pallas_call(
        _mlp_body,
        out_shape=jax.ShapeDtypeStruct((OUT, b), jnp.float32),
    )(ht, idx.reshape(1, b), tail_t, W2, b2.reshape(OUT, 1))
    return yt.T


# speedup vs baseline: 17.1113x; 1.0158x over previous
import functools

import jax
import jax.numpy as jnp
from jax import lax
from jax.experimental import pallas as pl
from jax.experimental.pallas import tpu as pltpu
from jax.experimental.pallas import tpu_sc as plsc

HIDDEN = 32
OUT = 16
CB = 768
BLK = HIDDEN * CB
V = 1000000
NB = 999936 // CB
VMAIN = NB * CB
NTAIL = V - VMAIN
FLAT_N = NB * BLK


def _sc_detile(embt):
    info = plsc.get_sparse_core_info()
    nc, ns = info.num_cores, info.num_subcores
    nw = nc * ns
    mesh = plsc.VectorSubcoreMesh(core_axis_name="c", subcore_axis_name="s")

    @functools.partial(
        pl.kernel,
        mesh=mesh,
        out_type=jax.ShapeDtypeStruct((FLAT_N,), jnp.float32),
        scratch_types=[
            pltpu.VMEM((HIDDEN, CB), jnp.float32),
            pltpu.VMEM((HIDDEN, CB), jnp.float32),
            pltpu.VMEM((HIDDEN, CB), jnp.float32),
            pltpu.VMEM((HIDDEN, CB), jnp.float32),
            pltpu.SemaphoreType.DMA,
            pltpu.SemaphoreType.DMA,
            pltpu.SemaphoreType.DMA,
            pltpu.SemaphoreType.DMA,
            pltpu.SemaphoreType.DMA,
            pltpu.SemaphoreType.DMA,
            pltpu.SemaphoreType.DMA,
            pltpu.SemaphoreType.DMA,
        ],
    )
    def detile_kernel(embt_hbm, flat_hbm, b0, b1, b2, b3,
                      r0s, r1s, r2s, r3s, w0s, w1s, w2s, w3s):
        wid = lax.axis_index("s") * nc + lax.axis_index("c")
        n_my = (NB - wid + nw - 1) // nw
        bufs = [b0, b1, b2, b3]
        rsems = [r0s, r1s, r2s, r3s]
        wsems = [w0s, w1s, w2s, w3s]

        def read_block(g, buf, rsem):
            c0 = pl.multiple_of(g * CB, 128)
            return pltpu.async_copy(embt_hbm.at[:, pl.ds(c0, CB)], buf, rsem)

        def write_descs(g, buf, wsem):
            o0 = pl.multiple_of(g * BLK, 8)
            return [
                pltpu.make_async_copy(
                    buf.at[c], flat_hbm.at[pl.ds(o0 + c * CB, CB)], wsem
                )
                for c in range(HIDDEN)
            ]

        def body(i, _):
            for k in range(4):
                g = wid + (4 * i + k) * nw

                @pl.when(i > 0)
                def _drain():
                    for cp in write_descs(g, bufs[k], wsems[k]):
                        cp.wait()

                read_block(g, bufs[k], rsems[k])
            for k in range(4):
                g = wid + (4 * i + k) * nw
                pltpu.make_async_copy(
                    embt_hbm.at[:, pl.ds(pl.multiple_of(g * CB, 128), CB)],
                    bufs[k],
                    rsems[k],
                ).wait()
                for cp in write_descs(g, bufs[k], wsems[k]):
                    cp.start()
            return _

        n4 = n_my // 4
        lax.fori_loop(0, n4, body, None)
        for k in range(4):
            @pl.when(n4 > 0)
            def _final_drain(k=k):
                g = wid + (4 * (n4 - 1) + k) * nw
                for cp in write_descs(g, bufs[k], wsems[k]):
                    cp.wait()

        @pl.when((n_my % 4) == 1)
        def _tail():
            g = wid + (n_my - 1) * nw
            read_block(g, bufs[0], rsems[0]).wait()
            for cp in write_descs(g, bufs[0], wsems[0]):
                cp.start()
            for cp in write_descs(g, bufs[0], wsems[0]):
                cp.wait()

    return detile_kernel(embt)


def _sc_gather_t(flat, pos0):
    info = plsc.get_sparse_core_info()
    nc, ns = info.num_cores, info.num_subcores
    nw = nc * ns
    b = pos0.shape[0]
    assert b % (8 * nw) == 0
    b_per_w = b // nw
    n_vec = b_per_w // 16
    mesh = plsc.VectorSubcoreMesh(core_axis_name="c", subcore_axis_name="s")

    @functools.partial(
        pl.kernel,
        mesh=mesh,
        out_type=jax.ShapeDtypeStruct((HIDDEN, b), jnp.float32),
        scratch_types=[
            pltpu.VMEM((b_per_w,), jnp.int32),
            pltpu.VMEM((HIDDEN, b_per_w), jnp.int32),
            pltpu.VMEM((HIDDEN, b_per_w), jnp.float32),
            pltpu.SemaphoreType.DMA,
        ],
        compiler_params=pltpu.CompilerParams(use_tc_tiling_on_sc=False),
    )
    def gather_kernel(flat_hbm, pos_hbm, out_hbm, pos_v, idxc_v, rows_v, sem):
        wid = lax.axis_index("s") * nc + lax.axis_index("c")
        base = wid * b_per_w
        pltpu.sync_copy(pos_hbm.at[pl.ds(base, b_per_w)], pos_v)

        def build(t, _):
            sl = pl.ds(t * 16, 16)
            acc = pos_v[sl]
            for c in range(HIDDEN):
                idxc_v[c, sl] = acc
                acc = acc + CB
            return _

        lax.fori_loop(0, n_vec, build, None)

        copies = [
            pltpu.async_copy(flat_hbm.at[idxc_v.at[c]], rows_v.at[c], sem)
            for c in range(HIDDEN)
        ]
        for cp in copies:
            cp.wait()
        pltpu.sync_copy(rows_v, out_hbm.at[:, pl.ds(base, b_per_w)])

    return gather_kernel(flat, pos0)


def _mlp_body(ht_ref, idx_ref, tail_ref, w_ref, b_ref, o_ref):
    ht = ht_ref[...]
    idx = idx_ref[...]
    tail_sel = idx - VMAIN
    onehot = jnp.where(
        lax.broadcasted_iota(jnp.int32, (NTAIL, idx.shape[1]), 0) == tail_sel,
        1.0,
        0.0,
    )
    htail = lax.dot_general(
        tail_ref[...], onehot, (((1,), (0,)), ((), ())),
        preferred_element_type=jnp.float32,
    )
    ht = jnp.where(idx >= VMAIN, htail, ht)
    h = jnp.maximum(ht, 0.0)
    o_ref[...] = (
        lax.dot_general(
            w_ref[...], h, (((1,), (0,)), ((), ())),
            preferred_element_type=jnp.float32,
        )
        + b_ref[...]
    )


def kernel(x, emb, W2, b2):
    b = x.shape[0]
    idx = x.reshape(b).astype(jnp.int32)
    flat = _sc_detile(emb.T)

    idx_c = jnp.minimum(idx, VMAIN - 1)
    g = idx_c // CB
    pos0 = g * BLK + (idx_c - g * CB)
    ht = _sc_gather_t(flat, pos0)

    tail_t = emb[VMAIN:].T
    yt = pl.pallas_call(
        _mlp_body,
        out_shape=jax.ShapeDtypeStruct((OUT, b), jnp.float32),
    )(ht, idx.reshape(1, b), tail_t, W2, b2.reshape(OUT, 1))
    return yt.T
